# Initial kernel scaffold; baseline (speedup 1.0000x reference)
import functools

import jax
import jax.numpy as jnp
from jax import lax
from jax.experimental import pallas as pl
from jax.experimental.pallas import tpu as pltpu
from jax.experimental.pallas import tpu_sc as plsc

N_N = 50000
N_H = 10000
NNZ = 800000
C = 64

CH = 512
CPT_A = 49
EPT_A = CPT_A * CH
RPAD = 32 * EPT_A
RROWS = RPAD // 128
CPT_B = 2 * CPT_A
EPT_B = CPT_B * CH

CNT_CH = 1024
CNT_PAD = 819200
CNT_EPT = CNT_PAD // 32
CNT_CPT = CNT_EPT // CNT_CH
SS_EPT = CNT_PAD // 16
SS_CPT = SS_EPT // CNT_CH

HN = N_H + 16
NN = N_N + 16
DBH = N_N // 2
DBA = DBH + 8


def _mesh():
    return plsc.VectorSubcoreMesh(core_axis_name="c", subcore_axis_name="s")


def _zero16():
    return jnp.zeros((16,), jnp.float32)


@functools.cache
def _sc_counts():
    @functools.partial(
        pl.kernel,
        out_type=(
            jax.ShapeDtypeStruct((32, HN), jnp.float32),
            jax.ShapeDtypeStruct((32, NN), jnp.float32),
        ),
        mesh=_mesh(),
        scratch_types=[
            pltpu.VMEM((HN,), jnp.float32),
            pltpu.VMEM((NN,), jnp.float32),
            pltpu.VMEM((CNT_CH,), jnp.int32),
            pltpu.VMEM((CNT_CH,), jnp.int32),
        ],
    )
    def k(hc_hbm, nc_hbm, de_p, dv_p, acc_de, acc_dv, hbuf, nbuf):
        c = lax.axis_index("c")
        s = lax.axis_index("s")
        w = c * 16 + s
        z = _zero16()
        ones = jnp.ones((16,), jnp.float32)

        @pl.loop(0, HN // 16)
        def _(i):
            acc_de[pl.ds(i * 16, 16)] = z

        @pl.loop(0, NN // 16)
        def _(i):
            acc_dv[pl.ds(i * 16, 16)] = z

        @pl.loop(0, CNT_CPT)
        def _(kk):
            b = w * CNT_EPT + kk * CNT_CH
            pltpu.sync_copy(hc_hbm.at[pl.ds(b, CNT_CH)], hbuf)
            pltpu.sync_copy(nc_hbm.at[pl.ds(b, CNT_CH)], nbuf)

            @pl.loop(0, CNT_CH // 16)
            def _(i):
                hv = hbuf[pl.ds(i * 16, 16)]
                plsc.addupdate_scatter(acc_de, [hv], ones)
                nv = nbuf[pl.ds(i * 16, 16)]
                plsc.addupdate_scatter(acc_dv, [nv], ones)

        pltpu.sync_copy(acc_de, de_p.at[w])
        pltpu.sync_copy(acc_dv, dv_p.at[w])

    return k


@functools.cache
def _sc_ssums():
    @functools.partial(
        pl.kernel,
        out_type=(
            jax.ShapeDtypeStruct((16, NN), jnp.float32),
            jax.ShapeDtypeStruct((16, HN), jnp.float32),
        ),
        mesh=_mesh(),
        scratch_types=[
            pltpu.VMEM((HN,), jnp.float32),
            pltpu.VMEM((NN,), jnp.float32),
            pltpu.VMEM((HN,), jnp.float32),
            pltpu.VMEM((NN,), jnp.float32),
            pltpu.VMEM((CNT_CH,), jnp.int32),
            pltpu.VMEM((CNT_CH,), jnp.int32),
        ],
    )
    def k(hc_hbm, nc_hbm, dea_hbm, dvb_hbm, s0_p, s1_p,
          tbl_s, tbl_b, acc_s, acc_b, hbuf, nbuf):
        c = lax.axis_index("c")
        s = lax.axis_index("s")
        z = _zero16()

        @pl.when(c == 0)
        def _():
            pltpu.sync_copy(dea_hbm, tbl_s)

            @pl.loop(0, NN // 16)
            def _(i):
                acc_b[pl.ds(i * 16, 16)] = z

            @pl.loop(0, SS_CPT)
            def _(kk):
                b = s * SS_EPT + kk * CNT_CH
                pltpu.sync_copy(hc_hbm.at[pl.ds(b, CNT_CH)], hbuf)
                pltpu.sync_copy(nc_hbm.at[pl.ds(b, CNT_CH)], nbuf)

                @pl.loop(0, CNT_CH // 16)
                def _(i):
                    hv = hbuf[pl.ds(i * 16, 16)]
                    nv = nbuf[pl.ds(i * 16, 16)]
                    val = plsc.load_gather(tbl_s, [hv])
                    plsc.addupdate_scatter(acc_b, [nv], val)

            pltpu.sync_copy(acc_b, s0_p.at[s])

        @pl.when(c == 1)
        def _():
            pltpu.sync_copy(dvb_hbm, tbl_b)

            @pl.loop(0, HN // 16)
            def _(i):
                acc_s[pl.ds(i * 16, 16)] = z

            @pl.loop(0, SS_CPT)
            def _(kk):
                b = s * SS_EPT + kk * CNT_CH
                pltpu.sync_copy(hc_hbm.at[pl.ds(b, CNT_CH)], hbuf)
                pltpu.sync_copy(nc_hbm.at[pl.ds(b, CNT_CH)], nbuf)

                @pl.loop(0, CNT_CH // 16)
                def _(i):
                    hv = hbuf[pl.ds(i * 16, 16)]
                    nv = nbuf[pl.ds(i * 16, 16)]
                    val = plsc.load_gather(tbl_b, [nv])
                    plsc.addupdate_scatter(acc_s, [hv], val)

            pltpu.sync_copy(acc_s, s1_p.at[s])

    return k


def _zero_rows(rows_v):
    z = _zero16()

    @pl.loop(0, CH)
    def _(i):
        for q in range(4):
            rows_v[i, pl.ds(q * 16, 16)] = z


@functools.cache
def _sc_seg_hedge():
    @functools.partial(
        pl.kernel,
        out_type=jax.ShapeDtypeStruct((2 * HN, C), jnp.float32),
        mesh=_mesh(),
        scratch_types=[
            pltpu.VMEM((4, 128), jnp.int32),
            pltpu.VMEM((4, 128), jnp.int32),
            pltpu.VMEM((CH, C), jnp.float32),
            pltpu.VMEM_SHARED((HN, C), jnp.float32),
            pltpu.SemaphoreType.DMA,
        ],
    )
    def k(in_hbm, src_hbm, dst_hbm, out_hbm, src_v, idx_v, rows_v, acc, sem):
        c = lax.axis_index("c")
        s = lax.axis_index("s")
        w = c * 16 + s

        _zero_rows(rows_v)
        st = s * (HN // 16)
        pltpu.sync_copy(rows_v, acc.at[pl.ds(st, CH)])
        pltpu.sync_copy(rows_v.at[pl.ds(0, HN // 16 - CH)],
                        acc.at[pl.ds(st + CH, HN // 16 - CH)])
        plsc.subcore_barrier()

        @pl.loop(0, CPT_A)
        def _(kk):
            rb = w * (EPT_A // 128) + kk * 4
            pltpu.sync_copy(src_hbm.at[pl.ds(rb, 4)], src_v)
            pltpu.sync_copy(dst_hbm.at[pl.ds(rb, 4)], idx_v)
            descs = [
                pltpu.async_copy(in_hbm.at[src_v.at[j]],
                                 rows_v.at[pl.ds(j * 128, 128)], sem)
                for j in range(4)
            ]
            for d in descs:
                d.wait()
            for j in range(4):
                pltpu.sync_copy(rows_v.at[pl.ds(j * 128, 128)],
                                acc.at[idx_v.at[j]], add=True)

        plsc.subcore_barrier()
        pltpu.sync_copy(acc.at[pl.ds(st, CH)],
                        out_hbm.at[pl.ds(c * HN + st, CH)])
        pltpu.sync_copy(acc.at[pl.ds(st + CH, HN // 16 - CH)],
                        out_hbm.at[pl.ds(c * HN + st + CH, HN // 16 - CH)])

    return k


@functools.cache
def _sc_seg_node():
    @functools.partial(
        pl.kernel,
        out_type=jax.ShapeDtypeStruct((N_N, C), jnp.float32),
        mesh=_mesh(),
        scratch_types=[
            pltpu.VMEM((4, 128), jnp.int32),
            pltpu.VMEM((4, 128), jnp.int32),
            pltpu.VMEM((CH, C), jnp.float32),
            pltpu.VMEM_SHARED((DBA, C), jnp.float32),
            pltpu.SemaphoreType.DMA,
        ],
    )
    def k(in_hbm, src_hbm, dst_hbm, out_hbm, src_v, idx_v, rows_v, acc, sem):
        c = lax.axis_index("c")
        s = lax.axis_index("s")
        lo = c * DBH

        _zero_rows(rows_v)
        st = s * (DBA // 16)
        for t in range(3):
            pltpu.sync_copy(rows_v, acc.at[pl.ds(st + t * CH, CH)])
        pltpu.sync_copy(rows_v.at[pl.ds(0, DBA // 16 - 3 * CH)],
                        acc.at[pl.ds(st + 3 * CH, DBA // 16 - 3 * CH)])
        plsc.subcore_barrier()

        @pl.loop(0, CPT_B)
        def _(kk):
            rb = s * (EPT_B // 128) + kk * 4
            pltpu.sync_copy(src_hbm.at[pl.ds(rb, 4)], src_v)
            pltpu.sync_copy(dst_hbm.at[pl.ds(rb, 4)], idx_v)
            for r in range(4):
                for t in range(8):
                    v = idx_v[r, pl.ds(t * 16, 16)]
                    li = v - lo
                    ok = (li >= 0) & (li < DBH)
                    idx_v[r, pl.ds(t * 16, 16)] = jnp.where(ok, li, DBH)
            descs = [
                pltpu.async_copy(in_hbm.at[src_v.at[j]],
                                 rows_v.at[pl.ds(j * 128, 128)], sem)
                for j in range(4)
            ]
            for d in descs:
                d.wait()
            for j in range(4):
                pltpu.sync_copy(rows_v.at[pl.ds(j * 128, 128)],
                                acc.at[idx_v.at[j]], add=True)

        plsc.subcore_barrier()
        st15 = s * 1563
        for t in range(3):
            pltpu.sync_copy(acc.at[pl.ds(st15 + t * CH, CH)],
                            out_hbm.at[pl.ds(lo + st15 + t * CH, CH)])

        @pl.when(s < 15)
        def _():
            pltpu.sync_copy(acc.at[pl.ds(st15 + 3 * CH, 27)],
                            out_hbm.at[pl.ds(lo + st15 + 3 * CH, 27)])

        @pl.when(s == 15)
        def _():
            pltpu.sync_copy(acc.at[pl.ds(st15 + 3 * CH, 19)],
                            out_hbm.at[pl.ds(lo + st15 + 3 * CH, 19)])

    return k


@functools.cache
def _tc_scales():
    def body(de_ref, dv_ref, dea_ref, dvb_ref):
        de = jnp.sum(de_ref[...], axis=0, keepdims=True)
        r = lax.rsqrt(de)
        dea_ref[...] = jnp.where(de > 0, r * r * r, 0.0)
        dv = jnp.sum(dv_ref[...], axis=0, keepdims=True)
        r2 = lax.rsqrt(dv)
        dvb_ref[...] = jnp.where(dv > 0, r2, 0.0)

    return pl.pallas_call(
        body,
        out_shape=(
            jax.ShapeDtypeStruct((1, HN), jnp.float32),
            jax.ShapeDtypeStruct((1, NN), jnp.float32),
        ),
    )


@functools.cache
def _tc_pre():
    blk = 2500

    def body(x_ref, sc_ref, w_ref, o_ref):
        o_ref[...] = jnp.dot(x_ref[...] * sc_ref[...], w_ref[...],
                             preferred_element_type=jnp.float32)

    return pl.pallas_call(
        body,
        grid=(N_N // blk,),
        in_specs=[
            pl.BlockSpec((blk, C), lambda i: (i, 0)),
            pl.BlockSpec((blk, 1), lambda i: (i, 0)),
            pl.BlockSpec((C, C), lambda i: (0, 0)),
        ],
        out_specs=pl.BlockSpec((blk, C), lambda i: (i, 0)),
        out_shape=jax.ShapeDtypeStruct((N_N, C), jnp.float32),
    )


@functools.cache
def _tc_hedge():
    blk = 2000

    def body(pa_ref, sp_ref, dea_ref, b_ref, w_ref, o_ref):
        seg = pa_ref[0] + pa_ref[1]
        s1 = jnp.sum(sp_ref[...], axis=1, keepdims=True)
        inv = jnp.where(s1 > 0, 1.0 / s1, 0.0)
        x1 = jnp.maximum(seg * inv + b_ref[...], 0.0)
        o_ref[...] = jnp.dot(x1 * dea_ref[...], w_ref[...],
                             preferred_element_type=jnp.float32)

    return pl.pallas_call(
        body,
        grid=(N_H // blk,),
        in_specs=[
            pl.BlockSpec((2, blk, C), lambda i: (0, i, 0)),
            pl.BlockSpec((blk, 16), lambda i: (i, 0)),
            pl.BlockSpec((blk, 1), lambda i: (i, 0)),
            pl.BlockSpec((1, C), lambda i: (0, 0)),
            pl.BlockSpec((C, C), lambda i: (0, 0)),
        ],
        out_specs=pl.BlockSpec((blk, C), lambda i: (i, 0)),
        out_shape=jax.ShapeDtypeStruct((N_H, C), jnp.float32),
    )


@functools.cache
def _tc_node():
    blk = 2500

    def body(seg_ref, sp_ref, b_ref, sc_ref, w_ref, o_ref):
        s0 = jnp.sum(sp_ref[...], axis=1, keepdims=True)
        inv = jnp.where(s0 > 0, 1.0 / s0, 0.0)
        x = jnp.maximum(seg_ref[...] * inv + b_ref[...], 0.0)
        o_ref[...] = jnp.dot(x * sc_ref[...], w_ref[...],
                             preferred_element_type=jnp.float32)

    return pl.pallas_call(
        body,
        grid=(N_N // blk,),
        in_specs=[
            pl.BlockSpec((blk, C), lambda i: (i, 0)),
            pl.BlockSpec((blk, 16), lambda i: (i, 0)),
            pl.BlockSpec((1, C), lambda i: (0, 0)),
            pl.BlockSpec((blk, 1), lambda i: (i, 0)),
            pl.BlockSpec((C, C), lambda i: (0, 0)),
        ],
        out_specs=pl.BlockSpec((blk, C), lambda i: (i, 0)),
        out_shape=jax.ShapeDtypeStruct((N_N, C), jnp.float32),
    )


@functools.cache
def _tc_final():
    blk = 2500
    ngrid = N_N // blk

    def body(seg_ref, sp_ref, b_ref, wl_ref, bl_ref, o_ref, pool_ref):
        s0 = jnp.sum(sp_ref[...], axis=1, keepdims=True)
        inv = jnp.where(s0 > 0, 1.0 / s0, 0.0)
        x = jnp.maximum(seg_ref[...] * inv + b_ref[...], 0.0)
        bm = jnp.max(x, axis=0, keepdims=True)
        i = pl.program_id(0)

        @pl.when(i == 0)
        def _():
            pool_ref[...] = bm

        @pl.when(i > 0)
        def _():
            pool_ref[...] = jnp.maximum(pool_ref[...], bm)

        @pl.when(i == ngrid - 1)
        def _():
            o_ref[...] = jnp.dot(pool_ref[...], wl_ref[...],
                                 preferred_element_type=jnp.float32) + bl_ref[...]

    return pl.pallas_call(
        body,
        grid=(ngrid,),
        in_specs=[
            pl.BlockSpec((blk, C), lambda i: (i, 0)),
            pl.BlockSpec((blk, 16), lambda i: (i, 0)),
            pl.BlockSpec((1, C), lambda i: (0, 0)),
            pl.BlockSpec((C, 1), lambda i: (0, 0)),
            pl.BlockSpec((1, 1), lambda i: (0, 0)),
        ],
        out_specs=pl.BlockSpec((1, 1), lambda i: (0, 0)),
        out_shape=jax.ShapeDtypeStruct((1, 1), jnp.float32),
        scratch_shapes=[pltpu.VMEM((1, C), jnp.float32)],
    )


def _pad_to(a, n, val):
    return jnp.concatenate(
        [a, jnp.full((n - a.shape[0],), val, a.dtype)])


def kernel(x_0, node_idx, hedge_idx,
           W01_1, b01_1, W10_1, b10_1,
           W01_2, b01_2, W10_2, b10_2,
           W_lin, b_lin):
    ni = node_idx.astype(jnp.int32)
    hi = hedge_idx.astype(jnp.int32)

    src_a = _pad_to(ni, RPAD, 0).reshape(RROWS, 128)
    dst_a = _pad_to(hi, RPAD, N_H).reshape(RROWS, 128)
    src_b = _pad_to(hi, RPAD, 0).reshape(RROWS, 128)
    dst_b = _pad_to(ni, RPAD, N_N).reshape(RROWS, 128)
    hc = _pad_to(hi, CNT_PAD, N_H)
    nc = _pad_to(ni, CNT_PAD, N_N)

    de_p, dv_p = _sc_counts()(hc, nc)
    dea_t, dvb_t = _tc_scales()(de_p, dv_p)
    s0_p, s1_p = _sc_ssums()(hc, nc, dea_t[0], dvb_t[0])

    s0t = s0_p[:, :N_N].T
    s1t = s1_p[:, :N_H].T
    dea_col = dea_t[0, :N_H].reshape(N_H, 1)
    dvb_col = dvb_t[0, :N_N].reshape(N_N, 1)
    b01_1r = b01_1.reshape(1, C)
    b10_1r = b10_1.reshape(1, C)
    b01_2r = b01_2.reshape(1, C)
    b10_2r = b10_2.reshape(1, C)

    m = _tc_pre()(x_0, dvb_col, W01_1)
    pa = _sc_seg_hedge()(m, src_a, dst_a).reshape(2, HN, C)[:, :N_H]
    m1 = _tc_hedge()(pa, s1t, dea_col, b01_1r, W10_1)
    segb = _sc_seg_node()(m1, src_b, dst_b)
    m2 = _tc_node()(segb, s0t, b10_1r, dvb_col, W01_2)
    pa2 = _sc_seg_hedge()(m2, src_a, dst_a).reshape(2, HN, C)[:, :N_H]
    m3 = _tc_hedge()(pa2, s1t, dea_col, b01_2r, W10_2)
    segb2 = _sc_seg_node()(m3, src_b, dst_b)
    out = _tc_final()(segb2, s0t, b10_2r, W_lin, b_lin.reshape(1, 1))
    return out.reshape(1)


# speedup vs baseline: 9.8315x; 9.8315x over previous
import functools

import jax
import jax.numpy as jnp
from jax import lax
from jax.experimental import pallas as pl
from jax.experimental.pallas import tpu as pltpu
from jax.experimental.pallas import tpu_sc as plsc

N_N = 50000
N_H = 10000
NNZ = 800000
C = 64

CH = 512
EPT_A = 25600
ITER_A = EPT_A // 1024
RPAD = 32 * EPT_A
RROWS = RPAD // 128
EPT_B = 2 * EPT_A
ITER_B = 2 * ITER_A

CNT_CH = 1024
CNT_PAD = 819200
CNT_EPT = CNT_PAD // 32
CNT_CPT = CNT_EPT // CNT_CH
SS_EPT = CNT_PAD // 16
SS_CPT = SS_EPT // CNT_CH

HN = 10112
NN = 50016
DBH = N_N // 2
DBA = 25088


def _mesh():
    return plsc.VectorSubcoreMesh(core_axis_name="c", subcore_axis_name="s")


def _zero16():
    return jnp.zeros((16,), jnp.float32)


@functools.cache
def _sc_counts():
    @functools.partial(
        pl.kernel,
        out_type=(
            jax.ShapeDtypeStruct((32 * HN,), jnp.float32),
            jax.ShapeDtypeStruct((32 * NN,), jnp.float32),
        ),
        mesh=_mesh(),
        compiler_params=pltpu.CompilerParams(needs_layout_passes=False, use_tc_tiling_on_sc=False),
        scratch_types=[
            pltpu.VMEM((HN,), jnp.float32),
            pltpu.VMEM((NN,), jnp.float32),
            pltpu.VMEM((CNT_CH,), jnp.int32),
            pltpu.VMEM((CNT_CH,), jnp.int32),
        ],
    )
    def k(hc_hbm, nc_hbm, de_p, dv_p, acc_de, acc_dv, hbuf, nbuf):
        c = lax.axis_index("c")
        s = lax.axis_index("s")
        w = c * 16 + s
        z = _zero16()
        ones = jnp.ones((16,), jnp.float32)

        @pl.loop(0, HN // 16)
        def _(i):
            acc_de[pl.ds(i * 16, 16)] = z

        @pl.loop(0, NN // 16)
        def _(i):
            acc_dv[pl.ds(i * 16, 16)] = z

        @pl.loop(0, CNT_CPT)
        def _(kk):
            b = w * CNT_EPT + kk * CNT_CH
            pltpu.sync_copy(hc_hbm.at[pl.ds(b, CNT_CH)], hbuf)
            pltpu.sync_copy(nc_hbm.at[pl.ds(b, CNT_CH)], nbuf)

            @pl.loop(0, CNT_CH // 16)
            def _(i):
                hv = hbuf[pl.ds(i * 16, 16)]
                plsc.addupdate_scatter(acc_de, [hv], ones)
                nv = nbuf[pl.ds(i * 16, 16)]
                plsc.addupdate_scatter(acc_dv, [nv], ones)

        pltpu.sync_copy(acc_de, de_p.at[pl.ds(w * HN, HN)])
        pltpu.sync_copy(acc_dv, dv_p.at[pl.ds(w * NN, NN)])

    return k


@functools.cache
def _sc_ssums():
    @functools.partial(
        pl.kernel,
        out_type=(
            jax.ShapeDtypeStruct((16 * NN,), jnp.float32),
            jax.ShapeDtypeStruct((16 * HN,), jnp.float32),
        ),
        mesh=_mesh(),
        compiler_params=pltpu.CompilerParams(needs_layout_passes=False, use_tc_tiling_on_sc=False),
        scratch_types=[
            pltpu.VMEM((HN,), jnp.float32),
            pltpu.VMEM((NN,), jnp.float32),
            pltpu.VMEM((HN,), jnp.float32),
            pltpu.VMEM((NN,), jnp.float32),
            pltpu.VMEM((CNT_CH,), jnp.int32),
            pltpu.VMEM((CNT_CH,), jnp.int32),
        ],
    )
    def k(hc_hbm, nc_hbm, dea_hbm, dvb_hbm, s0_p, s1_p,
          tbl_s, tbl_b, acc_s, acc_b, hbuf, nbuf):
        c = lax.axis_index("c")
        s = lax.axis_index("s")
        z = _zero16()

        @pl.when(c == 0)
        def _():
            pltpu.sync_copy(dea_hbm, tbl_s)

            @pl.loop(0, NN // 16)
            def _(i):
                acc_b[pl.ds(i * 16, 16)] = z

            @pl.loop(0, SS_CPT)
            def _(kk):
                b = s * SS_EPT + kk * CNT_CH
                pltpu.sync_copy(hc_hbm.at[pl.ds(b, CNT_CH)], hbuf)
                pltpu.sync_copy(nc_hbm.at[pl.ds(b, CNT_CH)], nbuf)

                @pl.loop(0, CNT_CH // 16)
                def _(i):
                    hv = hbuf[pl.ds(i * 16, 16)]
                    nv = nbuf[pl.ds(i * 16, 16)]
                    val = plsc.load_gather(tbl_s, [hv])
                    plsc.addupdate_scatter(acc_b, [nv], val)

            pltpu.sync_copy(acc_b, s0_p.at[pl.ds(s * NN, NN)])

        @pl.when(c == 1)
        def _():
            pltpu.sync_copy(dvb_hbm, tbl_b)

            @pl.loop(0, HN // 16)
            def _(i):
                acc_s[pl.ds(i * 16, 16)] = z

            @pl.loop(0, SS_CPT)
            def _(kk):
                b = s * SS_EPT + kk * CNT_CH
                pltpu.sync_copy(hc_hbm.at[pl.ds(b, CNT_CH)], hbuf)
                pltpu.sync_copy(nc_hbm.at[pl.ds(b, CNT_CH)], nbuf)

                @pl.loop(0, CNT_CH // 16)
                def _(i):
                    hv = hbuf[pl.ds(i * 16, 16)]
                    nv = nbuf[pl.ds(i * 16, 16)]
                    val = plsc.load_gather(tbl_b, [nv])
                    plsc.addupdate_scatter(acc_s, [hv], val)

            pltpu.sync_copy(acc_s, s1_p.at[pl.ds(s * HN, HN)])

    return k


def _zero_rows(rows_v, n):
    z = _zero16()

    @pl.loop(0, n)
    def _(i):
        for q in range(4):
            rows_v[i, pl.ds(q * 16, 16)] = z


@functools.cache
def _sc_seg_hedge():
    @functools.partial(
        pl.kernel,
        out_type=jax.ShapeDtypeStruct((2 * HN, C), jnp.float32),
        mesh=_mesh(),
        compiler_params=pltpu.CompilerParams(needs_layout_passes=False, use_tc_tiling_on_sc=False),
        scratch_types=[
            pltpu.VMEM((8, 128), jnp.int32),
            pltpu.VMEM((8, 128), jnp.int32),
            pltpu.VMEM((CH, C), jnp.float32),
            pltpu.VMEM_SHARED((HN, C), jnp.float32),
            pltpu.SemaphoreType.DMA,
        ],
    )
    def k(in_hbm, src_hbm, dst_hbm, out_hbm, src_v, idx_v, rows_v, acc, sem):
        c = lax.axis_index("c")
        s = lax.axis_index("s")
        w = c * 16 + s

        _zero_rows(rows_v, CH)
        st = s * (HN // 16)
        pltpu.sync_copy(rows_v, acc.at[pl.ds(st, CH)])
        pltpu.sync_copy(rows_v.at[pl.ds(0, HN // 16 - CH)],
                        acc.at[pl.ds(st + CH, HN // 16 - CH)])
        plsc.subcore_barrier()

        @pl.loop(0, ITER_A)
        def _(kk):
            rb = w * (EPT_A // 128) + kk * 8
            pltpu.sync_copy(src_hbm.at[pl.ds(rb, 8)], src_v)
            pltpu.sync_copy(dst_hbm.at[pl.ds(rb, 8)], idx_v)
            for h in range(2):
                descs = [
                    pltpu.async_copy(in_hbm.at[src_v.at[h * 4 + j]],
                                     rows_v.at[pl.ds(j * 128, 128)], sem)
                    for j in range(4)
                ]
                for d in descs:
                    d.wait()
                for j in range(4):
                    pltpu.sync_copy(rows_v.at[pl.ds(j * 128, 128)],
                                    acc.at[idx_v.at[h * 4 + j]], add=True)

        plsc.subcore_barrier()
        pltpu.sync_copy(acc.at[pl.ds(st, CH)],
                        out_hbm.at[pl.ds(c * HN + st, CH)])
        pltpu.sync_copy(acc.at[pl.ds(st + CH, HN // 16 - CH)],
                        out_hbm.at[pl.ds(c * HN + st + CH, HN // 16 - CH)])

    return k


@functools.cache
def _sc_seg_node():
    @functools.partial(
        pl.kernel,
        out_type=jax.ShapeDtypeStruct((N_N, C), jnp.float32),
        mesh=_mesh(),
        compiler_params=pltpu.CompilerParams(needs_layout_passes=False, use_tc_tiling_on_sc=False),
        scratch_types=[
            pltpu.VMEM((8, 128), jnp.int32),
            pltpu.VMEM((8, 128), jnp.int32),
            pltpu.VMEM((256, C), jnp.float32),
            pltpu.VMEM_SHARED((DBA, C), jnp.float32),
            pltpu.SemaphoreType.DMA,
        ],
    )
    def k(in_hbm, src_hbm, dst_hbm, out_hbm, src_v, idx_v, rows_v, acc, sem):
        c = lax.axis_index("c")
        s = lax.axis_index("s")
        lo = c * DBH

        _zero_rows(rows_v, 256)
        st = s * (DBA // 16)
        for t in range(6):
            pltpu.sync_copy(rows_v, acc.at[pl.ds(st + t * 256, 256)])
        pltpu.sync_copy(rows_v.at[pl.ds(0, 32)], acc.at[pl.ds(st + 1536, 32)])
        plsc.subcore_barrier()

        @pl.loop(0, ITER_B)
        def _(kk):
            rb = s * (EPT_B // 128) + kk * 8
            pltpu.sync_copy(src_hbm.at[pl.ds(rb, 8)], src_v)
            pltpu.sync_copy(dst_hbm.at[pl.ds(rb, 8)], idx_v)
            for r in range(8):
                for t in range(8):
                    v = idx_v[r, pl.ds(t * 16, 16)]
                    li = v - lo
                    ok = (li >= 0) & (li < DBH)
                    idx_v[r, pl.ds(t * 16, 16)] = jnp.where(ok, li, DBH)
            for h in range(4):
                descs = [
                    pltpu.async_copy(in_hbm.at[src_v.at[h * 2 + j]],
                                     rows_v.at[pl.ds(j * 128, 128)], sem)
                    for j in range(2)
                ]
                for d in descs:
                    d.wait()
                for j in range(2):
                    pltpu.sync_copy(rows_v.at[pl.ds(j * 128, 128)],
                                    acc.at[idx_v.at[h * 2 + j]], add=True)

        plsc.subcore_barrier()
        st15 = s * 1560
        for t in range(3):
            pltpu.sync_copy(acc.at[pl.ds(st15 + t * CH, CH)],
                            out_hbm.at[pl.ds(lo + st15 + t * CH, CH)])
        pltpu.sync_copy(acc.at[pl.ds(st15 + 3 * CH, 24)],
                        out_hbm.at[pl.ds(lo + st15 + 3 * CH, 24)])

        @pl.when(s == 0)
        def _():
            pltpu.sync_copy(acc.at[pl.ds(24960, 40)],
                            out_hbm.at[pl.ds(lo + 24960, 40)])

    return k


@functools.cache
def _tc_scales():
    def body(de_ref, dv_ref, dea_ref, dvb_ref):
        de = jnp.sum(de_ref[...], axis=0, keepdims=True)
        r = lax.rsqrt(de)
        dea_ref[...] = jnp.where(de > 0, r * r * r, 0.0)
        dv = jnp.sum(dv_ref[...], axis=0, keepdims=True)
        r2 = lax.rsqrt(dv)
        dvb_ref[...] = jnp.where(dv > 0, r2, 0.0)

    return pl.pallas_call(
        body,
        out_shape=(
            jax.ShapeDtypeStruct((1, HN), jnp.float32),
            jax.ShapeDtypeStruct((1, NN), jnp.float32),
        ),
    )


@functools.cache
def _tc_pre():
    blk = 2000

    def body(x_ref, sc_ref, w_ref, o_ref):
        o_ref[...] = jnp.dot(x_ref[...] * sc_ref[...], w_ref[...],
                             preferred_element_type=jnp.float32)

    return pl.pallas_call(
        body,
        grid=(N_N // blk,),
        in_specs=[
            pl.BlockSpec((blk, C), lambda i: (i, 0)),
            pl.BlockSpec((blk, 1), lambda i: (i, 0)),
            pl.BlockSpec((C, C), lambda i: (0, 0)),
        ],
        out_specs=pl.BlockSpec((blk, C), lambda i: (i, 0)),
        out_shape=jax.ShapeDtypeStruct((N_N, C), jnp.float32),
    )


@functools.cache
def _tc_hedge():
    blk = 2000

    def body(pa_ref, sp_ref, dea_ref, b_ref, w_ref, o_ref):
        seg = pa_ref[0] + pa_ref[1]
        s1 = jnp.sum(sp_ref[...], axis=1, keepdims=True)
        inv = jnp.where(s1 > 0, 1.0 / s1, 0.0)
        x1 = jnp.maximum(seg * inv + b_ref[...], 0.0)
        o_ref[...] = jnp.dot(x1 * dea_ref[...], w_ref[...],
                             preferred_element_type=jnp.float32)

    return pl.pallas_call(
        body,
        grid=(N_H // blk,),
        in_specs=[
            pl.BlockSpec((2, blk, C), lambda i: (0, i, 0)),
            pl.BlockSpec((blk, 16), lambda i: (i, 0)),
            pl.BlockSpec((blk, 1), lambda i: (i, 0)),
            pl.BlockSpec((1, C), lambda i: (0, 0)),
            pl.BlockSpec((C, C), lambda i: (0, 0)),
        ],
        out_specs=pl.BlockSpec((blk, C), lambda i: (i, 0)),
        out_shape=jax.ShapeDtypeStruct((N_H, C), jnp.float32),
    )


@functools.cache
def _tc_node():
    blk = 2000

    def body(seg_ref, sp_ref, b_ref, sc_ref, w_ref, o_ref):
        s0 = jnp.sum(sp_ref[...], axis=1, keepdims=True)
        inv = jnp.where(s0 > 0, 1.0 / s0, 0.0)
        x = jnp.maximum(seg_ref[...] * inv + b_ref[...], 0.0)
        o_ref[...] = jnp.dot(x * sc_ref[...], w_ref[...],
                             preferred_element_type=jnp.float32)

    return pl.pallas_call(
        body,
        grid=(N_N // blk,),
        in_specs=[
            pl.BlockSpec((blk, C), lambda i: (i, 0)),
            pl.BlockSpec((blk, 16), lambda i: (i, 0)),
            pl.BlockSpec((1, C), lambda i: (0, 0)),
            pl.BlockSpec((blk, 1), lambda i: (i, 0)),
            pl.BlockSpec((C, C), lambda i: (0, 0)),
        ],
        out_specs=pl.BlockSpec((blk, C), lambda i: (i, 0)),
        out_shape=jax.ShapeDtypeStruct((N_N, C), jnp.float32),
    )


@functools.cache
def _tc_final():
    blk = 2000
    ngrid = N_N // blk

    def body(seg_ref, sp_ref, b_ref, wl_ref, bl_ref, o_ref, pool_ref):
        s0 = jnp.sum(sp_ref[...], axis=1, keepdims=True)
        inv = jnp.where(s0 > 0, 1.0 / s0, 0.0)
        x = jnp.maximum(seg_ref[...] * inv + b_ref[...], 0.0)
        bm = jnp.max(x, axis=0, keepdims=True)
        i = pl.program_id(0)

        @pl.when(i == 0)
        def _():
            pool_ref[...] = bm

        @pl.when(i > 0)
        def _():
            pool_ref[...] = jnp.maximum(pool_ref[...], bm)

        @pl.when(i == ngrid - 1)
        def _():
            o_ref[...] = jnp.dot(pool_ref[...], wl_ref[...],
                                 preferred_element_type=jnp.float32) + bl_ref[...]

    return pl.pallas_call(
        body,
        grid=(ngrid,),
        in_specs=[
            pl.BlockSpec((blk, C), lambda i: (i, 0)),
            pl.BlockSpec((blk, 16), lambda i: (i, 0)),
            pl.BlockSpec((1, C), lambda i: (0, 0)),
            pl.BlockSpec((C, 1), lambda i: (0, 0)),
            pl.BlockSpec((1, 1), lambda i: (0, 0)),
        ],
        out_specs=pl.BlockSpec((1, 1), lambda i: (0, 0)),
        out_shape=jax.ShapeDtypeStruct((1, 1), jnp.float32),
        scratch_shapes=[pltpu.VMEM((1, C), jnp.float32)],
    )


def _pad_to(a, n, val):
    return jnp.concatenate(
        [a, jnp.full((n - a.shape[0],), val, a.dtype)])


def kernel(x_0, node_idx, hedge_idx,
           W01_1, b01_1, W10_1, b10_1,
           W01_2, b01_2, W10_2, b10_2,
           W_lin, b_lin):
    ni = node_idx.astype(jnp.int32)
    hi = hedge_idx.astype(jnp.int32)

    src_a = _pad_to(ni, RPAD, 0).reshape(RROWS, 128)
    dst_a = _pad_to(hi, RPAD, N_H).reshape(RROWS, 128)
    src_b = _pad_to(hi, RPAD, 0).reshape(RROWS, 128)
    dst_b = _pad_to(ni, RPAD, N_N).reshape(RROWS, 128)
    hc = _pad_to(hi, CNT_PAD, N_H)
    nc = _pad_to(ni, CNT_PAD, N_N)

    de_p, dv_p = _sc_counts()(hc, nc)
    dea_t, dvb_t = _tc_scales()(de_p.reshape(32, HN), dv_p.reshape(32, NN))
    s0_p, s1_p = _sc_ssums()(hc, nc, dea_t[0], dvb_t[0])
    s0_p = s0_p.reshape(16, NN)
    s1_p = s1_p.reshape(16, HN)

    s0t = s0_p[:, :N_N].T
    s1t = s1_p[:, :N_H].T
    dea_col = dea_t[0, :N_H].reshape(N_H, 1)
    dvb_col = dvb_t[0, :N_N].reshape(N_N, 1)
    b01_1r = b01_1.reshape(1, C)
    b10_1r = b10_1.reshape(1, C)
    b01_2r = b01_2.reshape(1, C)
    b10_2r = b10_2.reshape(1, C)

    m = _tc_pre()(x_0, dvb_col, W01_1)
    pa = _sc_seg_hedge()(m, src_a, dst_a).reshape(2, HN, C)[:, :N_H]
    m1 = _tc_hedge()(pa, s1t, dea_col, b01_1r, W10_1)
    segb = _sc_seg_node()(m1, src_b, dst_b)
    m2 = _tc_node()(segb, s0t, b10_1r, dvb_col, W01_2)
    pa2 = _sc_seg_hedge()(m2, src_a, dst_a).reshape(2, HN, C)[:, :N_H]
    m3 = _tc_hedge()(pa2, s1t, dea_col, b01_2r, W10_2)
    segb2 = _sc_seg_node()(m3, src_b, dst_b)
    out = _tc_final()(segb2, s0t, b10_2r, W_lin, b_lin.reshape(1, 1))
    return out.reshape(1)


# speedup vs baseline: 9.8455x; 1.0014x over previous
import functools

import jax
import jax.numpy as jnp
from jax import lax
from jax.experimental import pallas as pl
from jax.experimental.pallas import tpu as pltpu
from jax.experimental.pallas import tpu_sc as plsc

N_N = 50000
N_H = 10000
NNZ = 800000
C = 64

CH = 512
EPT_A = 25600
ITER_A = EPT_A // 1024
RPAD = 32 * EPT_A
RROWS = RPAD // 128
EPT_B = 2 * EPT_A
ITER_B = 2 * ITER_A

CNT_CH = 1024
CNT_PAD = 819200
CNT_EPT = CNT_PAD // 32
CNT_CPT = CNT_EPT // CNT_CH
SS_EPT = CNT_PAD // 16
SS_CPT = SS_EPT // CNT_CH

HN = 10112
NN = 50016
DBH = N_N // 2
DBA = 25088


def _mesh():
    return plsc.VectorSubcoreMesh(core_axis_name="c", subcore_axis_name="s")


def _zero16():
    return jnp.zeros((16,), jnp.float32)


@functools.cache
def _sc_counts():
    @functools.partial(
        pl.kernel,
        out_type=(
            jax.ShapeDtypeStruct((32 * HN,), jnp.float32),
            jax.ShapeDtypeStruct((32 * NN,), jnp.float32),
        ),
        mesh=_mesh(),
        compiler_params=pltpu.CompilerParams(needs_layout_passes=False, use_tc_tiling_on_sc=False),
        scratch_types=[
            pltpu.VMEM((HN,), jnp.float32),
            pltpu.VMEM((NN,), jnp.float32),
            pltpu.VMEM((CNT_CH,), jnp.int32),
            pltpu.VMEM((CNT_CH,), jnp.int32),
        ],
    )
    def k(hc_hbm, nc_hbm, de_p, dv_p, acc_de, acc_dv, hbuf, nbuf):
        c = lax.axis_index("c")
        s = lax.axis_index("s")
        w = c * 16 + s
        z = _zero16()
        ones = jnp.ones((16,), jnp.float32)

        @pl.loop(0, HN // 16)
        def _(i):
            acc_de[pl.ds(i * 16, 16)] = z

        @pl.loop(0, NN // 16)
        def _(i):
            acc_dv[pl.ds(i * 16, 16)] = z

        @pl.loop(0, CNT_CPT)
        def _(kk):
            b = w * CNT_EPT + kk * CNT_CH
            pltpu.sync_copy(hc_hbm.at[pl.ds(b, CNT_CH)], hbuf)
            pltpu.sync_copy(nc_hbm.at[pl.ds(b, CNT_CH)], nbuf)

            @pl.loop(0, CNT_CH // 16)
            def _(i):
                hv = hbuf[pl.ds(i * 16, 16)]
                plsc.addupdate_scatter(acc_de, [hv], ones)
                nv = nbuf[pl.ds(i * 16, 16)]
                plsc.addupdate_scatter(acc_dv, [nv], ones)

        pltpu.sync_copy(acc_de, de_p.at[pl.ds(w * HN, HN)])
        pltpu.sync_copy(acc_dv, dv_p.at[pl.ds(w * NN, NN)])

    return k


@functools.cache
def _sc_ssums():
    @functools.partial(
        pl.kernel,
        out_type=(
            jax.ShapeDtypeStruct((16 * NN,), jnp.float32),
            jax.ShapeDtypeStruct((16 * HN,), jnp.float32),
        ),
        mesh=_mesh(),
        compiler_params=pltpu.CompilerParams(needs_layout_passes=False, use_tc_tiling_on_sc=False),
        scratch_types=[
            pltpu.VMEM((HN,), jnp.float32),
            pltpu.VMEM((NN,), jnp.float32),
            pltpu.VMEM((HN,), jnp.float32),
            pltpu.VMEM((NN,), jnp.float32),
            pltpu.VMEM((CNT_CH,), jnp.int32),
            pltpu.VMEM((CNT_CH,), jnp.int32),
        ],
    )
    def k(hc_hbm, nc_hbm, dea_hbm, dvb_hbm, s0_p, s1_p,
          tbl_s, tbl_b, acc_s, acc_b, hbuf, nbuf):
        c = lax.axis_index("c")
        s = lax.axis_index("s")
        z = _zero16()

        @pl.when(c == 0)
        def _():
            pltpu.sync_copy(dea_hbm, tbl_s)

            @pl.loop(0, NN // 16)
            def _(i):
                acc_b[pl.ds(i * 16, 16)] = z

            @pl.loop(0, SS_CPT)
            def _(kk):
                b = s * SS_EPT + kk * CNT_CH
                pltpu.sync_copy(hc_hbm.at[pl.ds(b, CNT_CH)], hbuf)
                pltpu.sync_copy(nc_hbm.at[pl.ds(b, CNT_CH)], nbuf)

                @pl.loop(0, CNT_CH // 16)
                def _(i):
                    hv = hbuf[pl.ds(i * 16, 16)]
                    nv = nbuf[pl.ds(i * 16, 16)]
                    val = plsc.load_gather(tbl_s, [hv])
                    plsc.addupdate_scatter(acc_b, [nv], val)

            pltpu.sync_copy(acc_b, s0_p.at[pl.ds(s * NN, NN)])

        @pl.when(c == 1)
        def _():
            pltpu.sync_copy(dvb_hbm, tbl_b)

            @pl.loop(0, HN // 16)
            def _(i):
                acc_s[pl.ds(i * 16, 16)] = z

            @pl.loop(0, SS_CPT)
            def _(kk):
                b = s * SS_EPT + kk * CNT_CH
                pltpu.sync_copy(hc_hbm.at[pl.ds(b, CNT_CH)], hbuf)
                pltpu.sync_copy(nc_hbm.at[pl.ds(b, CNT_CH)], nbuf)

                @pl.loop(0, CNT_CH // 16)
                def _(i):
                    hv = hbuf[pl.ds(i * 16, 16)]
                    nv = nbuf[pl.ds(i * 16, 16)]
                    val = plsc.load_gather(tbl_b, [nv])
                    plsc.addupdate_scatter(acc_s, [hv], val)

            pltpu.sync_copy(acc_s, s1_p.at[pl.ds(s * HN, HN)])

    return k


def _zero_rows(rows_v, n):
    z = _zero16()

    @pl.loop(0, n)
    def _(i):
        for q in range(4):
            rows_v[i, pl.ds(q * 16, 16)] = z


@functools.cache
def _sc_seg_hedge():
    @functools.partial(
        pl.kernel,
        out_type=jax.ShapeDtypeStruct((2 * HN, C), jnp.float32),
        mesh=_mesh(),
        compiler_params=pltpu.CompilerParams(needs_layout_passes=False, use_tc_tiling_on_sc=False),
        scratch_types=[
            pltpu.VMEM((40, 128), jnp.int32),
            pltpu.VMEM((40, 128), jnp.int32),
            pltpu.VMEM((256, C), jnp.float32),
            pltpu.VMEM_SHARED((HN, C), jnp.float32),
            pltpu.SemaphoreType.DMA,
        ],
    )
    def k(in_hbm, src_hbm, dst_hbm, out_hbm, src_v, idx_v, rows_v, acc, sem):
        c = lax.axis_index("c")
        s = lax.axis_index("s")
        w = c * 16 + s

        _zero_rows(rows_v, 256)
        st = s * (HN // 16)
        for t in range(2):
            pltpu.sync_copy(rows_v, acc.at[pl.ds(st + t * 256, 256)])
        pltpu.sync_copy(rows_v.at[pl.ds(0, 120)], acc.at[pl.ds(st + 512, 120)])
        plsc.subcore_barrier()

        @pl.loop(0, ITER_A // 5)
        def _(u):
            rb = w * (EPT_A // 128) + u * 40
            pltpu.sync_copy(src_hbm.at[pl.ds(rb, 40)], src_v)
            pltpu.sync_copy(dst_hbm.at[pl.ds(rb, 40)], idx_v)
            _ring40(in_hbm, src_v, idx_v, rows_v, acc, sem)

        plsc.subcore_barrier()
        pltpu.sync_copy(acc.at[pl.ds(st, CH)],
                        out_hbm.at[pl.ds(c * HN + st, CH)])
        pltpu.sync_copy(acc.at[pl.ds(st + CH, HN // 16 - CH)],
                        out_hbm.at[pl.ds(c * HN + st + CH, HN // 16 - CH)])

    return k


def _ring40(in_hbm, src_v, idx_v, rows_v, acc, sem):
    pltpu.async_copy(in_hbm.at[src_v.at[0]], rows_v.at[pl.ds(0, 128)], sem)

    @pl.loop(0, 40)
    def _(p):
        b = (p % 2) * 128
        nb = 128 - b

        @pl.when(p < 39)
        def _():
            pltpu.async_copy(in_hbm.at[src_v.at[p + 1]],
                             rows_v.at[pl.ds(nb, 128)], sem)

        pltpu.make_async_copy(in_hbm.at[src_v.at[p]],
                              rows_v.at[pl.ds(b, 128)], sem).wait()
        pltpu.sync_copy(rows_v.at[pl.ds(b, 128)], acc.at[idx_v.at[p]],
                        add=True)


@functools.cache
def _sc_seg_node():
    @functools.partial(
        pl.kernel,
        out_type=jax.ShapeDtypeStruct((N_N, C), jnp.float32),
        mesh=_mesh(),
        compiler_params=pltpu.CompilerParams(needs_layout_passes=False, use_tc_tiling_on_sc=False),
        scratch_types=[
            pltpu.VMEM((40, 128), jnp.int32),
            pltpu.VMEM((40, 128), jnp.int32),
            pltpu.VMEM((256, C), jnp.float32),
            pltpu.VMEM_SHARED((DBA, C), jnp.float32),
            pltpu.SemaphoreType.DMA,
        ],
    )
    def k(in_hbm, src_hbm, dlo_hbm, dhi_hbm, out_hbm,
          src_v, idx_v, rows_v, acc, sem):
        c = lax.axis_index("c")
        s = lax.axis_index("s")
        lo = c * DBH

        _zero_rows(rows_v, 256)
        st = s * (DBA // 16)
        for t in range(6):
            pltpu.sync_copy(rows_v, acc.at[pl.ds(st + t * 256, 256)])
        pltpu.sync_copy(rows_v.at[pl.ds(0, 32)], acc.at[pl.ds(st + 1536, 32)])
        plsc.subcore_barrier()

        def main(dst_hbm):
            @pl.loop(0, ITER_B // 5)
            def _(u):
                rb = s * (EPT_B // 128) + u * 40
                pltpu.sync_copy(src_hbm.at[pl.ds(rb, 40)], src_v)
                pltpu.sync_copy(dst_hbm.at[pl.ds(rb, 40)], idx_v)
                _ring40(in_hbm, src_v, idx_v, rows_v, acc, sem)

        @pl.when(c == 0)
        def _():
            main(dlo_hbm)

        @pl.when(c == 1)
        def _():
            main(dhi_hbm)

        plsc.subcore_barrier()
        st15 = s * 1560
        for t in range(3):
            pltpu.sync_copy(acc.at[pl.ds(st15 + t * CH, CH)],
                            out_hbm.at[pl.ds(lo + st15 + t * CH, CH)])
        pltpu.sync_copy(acc.at[pl.ds(st15 + 3 * CH, 24)],
                        out_hbm.at[pl.ds(lo + st15 + 3 * CH, 24)])

        @pl.when(s == 0)
        def _():
            pltpu.sync_copy(acc.at[pl.ds(24960, 40)],
                            out_hbm.at[pl.ds(lo + 24960, 40)])

    return k


@functools.cache
def _tc_scales():
    def body(de_ref, dv_ref, dea_ref, dvb_ref):
        de = jnp.sum(de_ref[...], axis=0, keepdims=True)
        r = lax.rsqrt(de)
        dea_ref[...] = jnp.where(de > 0, r * r * r, 0.0)
        dv = jnp.sum(dv_ref[...], axis=0, keepdims=True)
        r2 = lax.rsqrt(dv)
        dvb_ref[...] = jnp.where(dv > 0, r2, 0.0)

    return pl.pallas_call(
        body,
        out_shape=(
            jax.ShapeDtypeStruct((1, HN), jnp.float32),
            jax.ShapeDtypeStruct((1, NN), jnp.float32),
        ),
    )


@functools.cache
def _tc_pre():
    blk = 2000

    def body(x_ref, sc_ref, w_ref, o_ref):
        o_ref[...] = jnp.dot(x_ref[...] * sc_ref[...], w_ref[...],
                             preferred_element_type=jnp.float32)

    return pl.pallas_call(
        body,
        grid=(N_N // blk,),
        in_specs=[
            pl.BlockSpec((blk, C), lambda i: (i, 0)),
            pl.BlockSpec((blk, 1), lambda i: (i, 0)),
            pl.BlockSpec((C, C), lambda i: (0, 0)),
        ],
        out_specs=pl.BlockSpec((blk, C), lambda i: (i, 0)),
        out_shape=jax.ShapeDtypeStruct((N_N, C), jnp.float32),
    )


@functools.cache
def _tc_hedge():
    blk = 2000

    def body(pa_ref, sp_ref, dea_ref, b_ref, w_ref, o_ref):
        seg = pa_ref[0] + pa_ref[1]
        s1 = jnp.sum(sp_ref[...], axis=1, keepdims=True)
        inv = jnp.where(s1 > 0, 1.0 / s1, 0.0)
        x1 = jnp.maximum(seg * inv + b_ref[...], 0.0)
        o_ref[...] = jnp.dot(x1 * dea_ref[...], w_ref[...],
                             preferred_element_type=jnp.float32)

    return pl.pallas_call(
        body,
        grid=(N_H // blk,),
        in_specs=[
            pl.BlockSpec((2, blk, C), lambda i: (0, i, 0)),
            pl.BlockSpec((blk, 16), lambda i: (i, 0)),
            pl.BlockSpec((blk, 1), lambda i: (i, 0)),
            pl.BlockSpec((1, C), lambda i: (0, 0)),
            pl.BlockSpec((C, C), lambda i: (0, 0)),
        ],
        out_specs=pl.BlockSpec((blk, C), lambda i: (i, 0)),
        out_shape=jax.ShapeDtypeStruct((N_H, C), jnp.float32),
    )


@functools.cache
def _tc_node():
    blk = 2000

    def body(seg_ref, sp_ref, b_ref, sc_ref, w_ref, o_ref):
        s0 = jnp.sum(sp_ref[...], axis=1, keepdims=True)
        inv = jnp.where(s0 > 0, 1.0 / s0, 0.0)
        x = jnp.maximum(seg_ref[...] * inv + b_ref[...], 0.0)
        o_ref[...] = jnp.dot(x * sc_ref[...], w_ref[...],
                             preferred_element_type=jnp.float32)

    return pl.pallas_call(
        body,
        grid=(N_N // blk,),
        in_specs=[
            pl.BlockSpec((blk, C), lambda i: (i, 0)),
            pl.BlockSpec((blk, 16), lambda i: (i, 0)),
            pl.BlockSpec((1, C), lambda i: (0, 0)),
            pl.BlockSpec((blk, 1), lambda i: (i, 0)),
            pl.BlockSpec((C, C), lambda i: (0, 0)),
        ],
        out_specs=pl.BlockSpec((blk, C), lambda i: (i, 0)),
        out_shape=jax.ShapeDtypeStruct((N_N, C), jnp.float32),
    )


@functools.cache
def _tc_final():
    blk = 2000
    ngrid = N_N // blk

    def body(seg_ref, sp_ref, b_ref, wl_ref, bl_ref, o_ref, pool_ref):
        s0 = jnp.sum(sp_ref[...], axis=1, keepdims=True)
        inv = jnp.where(s0 > 0, 1.0 / s0, 0.0)
        x = jnp.maximum(seg_ref[...] * inv + b_ref[...], 0.0)
        bm = jnp.max(x, axis=0, keepdims=True)
        i = pl.program_id(0)

        @pl.when(i == 0)
        def _():
            pool_ref[...] = bm

        @pl.when(i > 0)
        def _():
            pool_ref[...] = jnp.maximum(pool_ref[...], bm)

        @pl.when(i == ngrid - 1)
        def _():
            o_ref[...] = jnp.dot(pool_ref[...], wl_ref[...],
                                 preferred_element_type=jnp.float32) + bl_ref[...]

    return pl.pallas_call(
        body,
        grid=(ngrid,),
        in_specs=[
            pl.BlockSpec((blk, C), lambda i: (i, 0)),
            pl.BlockSpec((blk, 16), lambda i: (i, 0)),
            pl.BlockSpec((1, C), lambda i: (0, 0)),
            pl.BlockSpec((C, 1), lambda i: (0, 0)),
            pl.BlockSpec((1, 1), lambda i: (0, 0)),
        ],
        out_specs=pl.BlockSpec((1, 1), lambda i: (0, 0)),
        out_shape=jax.ShapeDtypeStruct((1, 1), jnp.float32),
        scratch_shapes=[pltpu.VMEM((1, C), jnp.float32)],
    )


def _pad_to(a, n, val):
    return jnp.concatenate(
        [a, jnp.full((n - a.shape[0],), val, a.dtype)])


def kernel(x_0, node_idx, hedge_idx,
           W01_1, b01_1, W10_1, b10_1,
           W01_2, b01_2, W10_2, b10_2,
           W_lin, b_lin):
    ni = node_idx.astype(jnp.int32)
    hi = hedge_idx.astype(jnp.int32)

    src_a = _pad_to(ni, RPAD, 0).reshape(RROWS, 128)
    dst_a = _pad_to(hi, RPAD, N_H).reshape(RROWS, 128)
    src_b = _pad_to(hi, RPAD, 0).reshape(RROWS, 128)
    nip = _pad_to(ni, RPAD, N_N)
    dst_blo = jnp.where(nip < DBH, nip, DBH).reshape(RROWS, 128)
    nih = nip - DBH
    dst_bhi = jnp.where((nih >= 0) & (nih < DBH), nih, DBH).reshape(RROWS, 128)
    hc = _pad_to(hi, CNT_PAD, N_H)
    nc = _pad_to(ni, CNT_PAD, N_N)

    de_p, dv_p = _sc_counts()(hc, nc)
    dea_t, dvb_t = _tc_scales()(de_p.reshape(32, HN), dv_p.reshape(32, NN))
    s0_p, s1_p = _sc_ssums()(hc, nc, dea_t[0], dvb_t[0])
    s0_p = s0_p.reshape(16, NN)
    s1_p = s1_p.reshape(16, HN)

    s0t = s0_p[:, :N_N].T
    s1t = s1_p[:, :N_H].T
    dea_col = dea_t[0, :N_H].reshape(N_H, 1)
    dvb_col = dvb_t[0, :N_N].reshape(N_N, 1)
    b01_1r = b01_1.reshape(1, C)
    b10_1r = b10_1.reshape(1, C)
    b01_2r = b01_2.reshape(1, C)
    b10_2r = b10_2.reshape(1, C)

    m = _tc_pre()(x_0, dvb_col, W01_1)
    pa = _sc_seg_hedge()(m, src_a, dst_a).reshape(2, HN, C)[:, :N_H]
    m1 = _tc_hedge()(pa, s1t, dea_col, b01_1r, W10_1)
    segb = _sc_seg_node()(m1, src_b, dst_blo, dst_bhi)
    m2 = _tc_node()(segb, s0t, b10_1r, dvb_col, W01_2)
    pa2 = _sc_seg_hedge()(m2, src_a, dst_a).reshape(2, HN, C)[:, :N_H]
    m3 = _tc_hedge()(pa2, s1t, dea_col, b01_2r, W10_2)
    segb2 = _sc_seg_node()(m3, src_b, dst_blo, dst_bhi)
    out = _tc_final()(segb2, s0t, b10_2r, W_lin, b_lin.reshape(1, 1))
    return out.reshape(1)
